# Initial kernel scaffold; baseline (speedup 1.0000x reference)
import jax
import jax.numpy as jnp
from jax.experimental import pallas as pl


def kernel(x, lengths, emb, W1, b1, W2, b2):
    raise NotImplementedError("write your pallas kernel here")


# speedup vs baseline: 8.1997x; 8.1997x over previous
import functools

import jax
import jax.numpy as jnp
from jax import lax
from jax.experimental import pallas as pl
from jax.experimental.pallas import tpu as pltpu
from jax.experimental.pallas import tpu_sc as plsc

B, L = 4096, 200
DIM = 64
HIDDEN, OUT = 1000, 10

NC, NS, LANES = 2, 16, 16
NW = NC * NS
SPB = B // NW
LH = L // 2
NCH = DIM // LANES


def _pool_body(x_hbm, emb_hbm, out_hbm, idx_v, rows_v, out_v, sem):
    wid = lax.axis_index("s") * NC + lax.axis_index("c")
    base = wid * SPB
    pltpu.sync_copy(x_hbm.at[pl.ds(base, SPB)], idx_v)

    def sample_body(i, carry):
        c0 = pltpu.async_copy(emb_hbm.at[idx_v.at[i, 0]],
                              rows_v.at[pl.ds(0, LH)], sem)
        c1 = pltpu.async_copy(emb_hbm.at[idx_v.at[i, 1]],
                              rows_v.at[pl.ds(LH, LH)], sem)
        c0.wait()
        c1.wait()

        def row_body(r, acc):
            return tuple(acc[c] + rows_v[r, pl.ds(c * LANES, LANES)]
                         for c in range(NCH))

        zero = jnp.zeros((LANES,), jnp.float32)
        acc = lax.fori_loop(0, L, row_body, (zero,) * NCH)
        for c in range(NCH):
            out_v[i, pl.ds(c * LANES, LANES)] = acc[c]
        return carry

    lax.fori_loop(0, SPB, sample_body, 0)
    pltpu.sync_copy(out_v, out_hbm.at[pl.ds(base, SPB)])


@functools.partial(jax.jit, static_argnums=())
def _pool(x3, emb):
    mesh = plsc.VectorSubcoreMesh(core_axis_name="c", subcore_axis_name="s",
                                  num_cores=NC, num_subcores=NS)
    return pl.kernel(
        _pool_body,
        out_type=jax.ShapeDtypeStruct((B, DIM), jnp.float32),
        mesh=mesh,
        scratch_types=[
            pltpu.VMEM((SPB, 2, LH), jnp.int32),
            pltpu.VMEM((L, DIM), jnp.float32),
            pltpu.VMEM((SPB, DIM), jnp.float32),
            pltpu.SemaphoreType.DMA,
        ],
        compiler_params=pltpu.CompilerParams(use_tc_tiling_on_sc=False),
    )(x3, emb)


def _mlp_body(rep_ref, len_ref, W1_ref, b1_ref, W2_ref, b2_ref, out_ref):
    inv = 1.0 / len_ref[...].astype(jnp.float32)
    r = rep_ref[...] * inv
    h = jnp.dot(r, W1_ref[...], preferred_element_type=jnp.float32)
    h = jnp.maximum(h + b1_ref[...], 0.0)
    out_ref[...] = (jnp.dot(h, W2_ref[...], preferred_element_type=jnp.float32)
                    + b2_ref[...])


MLP_BLK = 512


def _mlp(rep, lengths2, W1, b1r, W2, b2r):
    grid = (B // MLP_BLK,)
    return pl.pallas_call(
        _mlp_body,
        grid=grid,
        in_specs=[
            pl.BlockSpec((MLP_BLK, DIM), lambda i: (i, 0)),
            pl.BlockSpec((MLP_BLK, 1), lambda i: (i, 0)),
            pl.BlockSpec((DIM, HIDDEN), lambda i: (0, 0)),
            pl.BlockSpec((1, HIDDEN), lambda i: (0, 0)),
            pl.BlockSpec((HIDDEN, OUT), lambda i: (0, 0)),
            pl.BlockSpec((1, OUT), lambda i: (0, 0)),
        ],
        out_specs=pl.BlockSpec((MLP_BLK, OUT), lambda i: (i, 0)),
        out_shape=jax.ShapeDtypeStruct((B, OUT), jnp.float32),
    )(rep, lengths2, W1, b1r, W2, b2r)


def kernel(x, lengths, emb, W1, b1, W2, b2):
    x3 = x.astype(jnp.int32).reshape(B, 2, LH)
    sums = _pool(x3, emb)
    return _mlp(sums, lengths.reshape(B, 1), W1, b1.reshape(1, HIDDEN),
                W2, b2.reshape(1, OUT))


# speedup vs baseline: 12.8355x; 1.5654x over previous
import functools

import jax
import jax.numpy as jnp
from jax import lax
from jax.experimental import pallas as pl
from jax.experimental.pallas import tpu as pltpu
from jax.experimental.pallas import tpu_sc as plsc

B, L = 4096, 200
DIM = 64
HIDDEN, OUT = 1000, 10

NC, NS, LANES = 2, 16, 16
NW = NC * NS
SPB = B // NW
LH = L // 2
NCH = DIM // LANES


UNROLL = 8


def _pool_body(x_hbm, emb_hbm, out_hbm, idx_v, rows0, rows1, out_v,
               sem0, sem1):
    wid = lax.axis_index("s") * NC + lax.axis_index("c")
    base = wid * SPB
    pltpu.sync_copy(x_hbm.at[pl.ds(base, SPB)], idx_v)

    def fire(s, buf, sem):
        pltpu.async_copy(emb_hbm.at[idx_v.at[s, 0]],
                         buf.at[pl.ds(0, LH)], sem)
        pltpu.async_copy(emb_hbm.at[idx_v.at[s, 1]],
                         buf.at[pl.ds(LH, LH)], sem)

    def drain(buf, sem):
        pltpu.make_async_copy(emb_hbm.at[idx_v.at[0, 0]],
                              buf.at[pl.ds(0, LH)], sem).wait()
        pltpu.make_async_copy(emb_hbm.at[idx_v.at[0, 1]],
                              buf.at[pl.ds(LH, LH)], sem).wait()

    zero = jnp.zeros((LANES,), jnp.float32)

    def sumbuf(buf, s):
        def row_body(rr, accs):
            accs = list(accs)
            for u in range(UNROLL):
                r = rr * UNROLL + u
                p = (u & 1) * NCH
                for c in range(NCH):
                    accs[p + c] = accs[p + c] + buf[r, pl.ds(c * LANES, LANES)]
            return tuple(accs)

        accs = lax.fori_loop(0, L // UNROLL, row_body, (zero,) * (2 * NCH))
        for c in range(NCH):
            out_v[s, pl.ds(c * LANES, LANES)] = accs[c] + accs[NCH + c]

    fire(0, rows0, sem0)

    def pair_body(g, carry):
        s0 = 2 * g
        fire(s0 + 1, rows1, sem1)
        drain(rows0, sem0)
        sumbuf(rows0, s0)
        fire(jnp.minimum(s0 + 2, SPB - 1), rows0, sem0)
        drain(rows1, sem1)
        sumbuf(rows1, s0 + 1)
        return carry

    lax.fori_loop(0, SPB // 2, pair_body, 0)
    drain(rows0, sem0)
    pltpu.sync_copy(out_v, out_hbm.at[pl.ds(base, SPB)])


@functools.partial(jax.jit, static_argnums=())
def _pool(x3, emb):
    mesh = plsc.VectorSubcoreMesh(core_axis_name="c", subcore_axis_name="s",
                                  num_cores=NC, num_subcores=NS)
    return pl.kernel(
        _pool_body,
        out_type=jax.ShapeDtypeStruct((B, DIM), jnp.float32),
        mesh=mesh,
        scratch_types=[
            pltpu.VMEM((SPB, 2, LH), jnp.int32),
            pltpu.VMEM((L, DIM), jnp.float32),
            pltpu.VMEM((L, DIM), jnp.float32),
            pltpu.VMEM((SPB, DIM), jnp.float32),
            pltpu.SemaphoreType.DMA,
            pltpu.SemaphoreType.DMA,
        ],
        compiler_params=pltpu.CompilerParams(use_tc_tiling_on_sc=False),
    )(x3, emb)


def _mlp_body(rep_ref, len_ref, W1_ref, b1_ref, W2_ref, b2_ref, out_ref):
    inv = 1.0 / len_ref[...].astype(jnp.float32)
    r = rep_ref[...] * inv
    h = jnp.dot(r, W1_ref[...], preferred_element_type=jnp.float32)
    h = jnp.maximum(h + b1_ref[...], 0.0)
    out_ref[...] = (jnp.dot(h, W2_ref[...], preferred_element_type=jnp.float32)
                    + b2_ref[...])


MLP_BLK = 512


def _mlp(rep, lengths2, W1, b1r, W2, b2r):
    grid = (B // MLP_BLK,)
    return pl.pallas_call(
        _mlp_body,
        grid=grid,
        in_specs=[
            pl.BlockSpec((MLP_BLK, DIM), lambda i: (i, 0)),
            pl.BlockSpec((MLP_BLK, 1), lambda i: (i, 0)),
            pl.BlockSpec((DIM, HIDDEN), lambda i: (0, 0)),
            pl.BlockSpec((1, HIDDEN), lambda i: (0, 0)),
            pl.BlockSpec((HIDDEN, OUT), lambda i: (0, 0)),
            pl.BlockSpec((1, OUT), lambda i: (0, 0)),
        ],
        out_specs=pl.BlockSpec((MLP_BLK, OUT), lambda i: (i, 0)),
        out_shape=jax.ShapeDtypeStruct((B, OUT), jnp.float32),
    )(rep, lengths2, W1, b1r, W2, b2r)


def kernel(x, lengths, emb, W1, b1, W2, b2):
    x3 = x.astype(jnp.int32).reshape(B, 2, LH)
    sums = _pool(x3, emb)
    return _mlp(sums, lengths.reshape(B, 1), W1, b1.reshape(1, HIDDEN),
                W2, b2.reshape(1, OUT))


# speedup vs baseline: 16.7466x; 1.3047x over previous
import functools

import jax
import jax.numpy as jnp
from jax import lax
from jax.experimental import pallas as pl
from jax.experimental.pallas import tpu as pltpu
from jax.experimental.pallas import tpu_sc as plsc

B, L = 4096, 200
DIM = 64
HIDDEN, OUT = 1000, 10

NC, NS, LANES = 2, 16, 16
NW = NC * NS
SPB = B // NW
LA, LB = 104, 96
NCH = DIM // LANES
NBUF = 4
UNROLL = 8


def _pool_body(x_hbm, emb_hbm, out_hbm, idx_v, rows, out_v, sems):
    wid = lax.axis_index("s") * NC + lax.axis_index("c")
    base = wid * SPB
    pltpu.sync_copy(x_hbm.at[pl.ds(base * L, SPB * L)], idx_v)

    def fire(s, b):
        off = s * L
        pltpu.async_copy(emb_hbm.at[idx_v.at[pl.ds(off, LA)]],
                         rows[b].at[pl.ds(0, LA)], sems[b])
        pltpu.async_copy(emb_hbm.at[idx_v.at[pl.ds(off + LA, LB)]],
                         rows[b].at[pl.ds(LA, LB)], sems[b])

    def drain(b):
        pltpu.make_async_copy(emb_hbm.at[idx_v.at[pl.ds(0, LA)]],
                              rows[b].at[pl.ds(0, LA)], sems[b]).wait()
        pltpu.make_async_copy(emb_hbm.at[idx_v.at[pl.ds(0, LB)]],
                              rows[b].at[pl.ds(LA, LB)], sems[b]).wait()

    zero = jnp.zeros((LANES,), jnp.float32)

    def sumbuf(b, s):
        buf = rows[b]

        def row_body(rr, accs):
            accs = list(accs)
            for u in range(UNROLL):
                r = rr * UNROLL + u
                p = (u & 1) * NCH
                for c in range(NCH):
                    accs[p + c] = accs[p + c] + buf[r, pl.ds(c * LANES, LANES)]
            return tuple(accs)

        accs = lax.fori_loop(0, L // UNROLL, row_body, (zero,) * (2 * NCH))
        for c in range(NCH):
            out_v[pl.ds(s * DIM + c * LANES, LANES)] = accs[c] + accs[NCH + c]

    for b in range(NBUF - 1):
        fire(jnp.int32(b), b)

    def quad_body(g, carry):
        s0 = 4 * g
        for b in range(NBUF):
            s = s0 + b
            fire(jnp.minimum(s + NBUF - 1, SPB - 1), (b + NBUF - 1) % NBUF)
            drain(b)
            sumbuf(b, s)
        return carry

    lax.fori_loop(0, SPB // NBUF, quad_body, 0)
    for b in range(NBUF - 1):
        drain(b)
    pltpu.sync_copy(out_v, out_hbm.at[pl.ds(base * DIM, SPB * DIM)])


def _pool(x1, emb):
    mesh = plsc.VectorSubcoreMesh(core_axis_name="c", subcore_axis_name="s",
                                  num_cores=NC, num_subcores=NS)

    def body2(x_hbm, emb_hbm, out_hbm, idx_v, r0, r1, r2, r3, out_v,
              s0, s1, s2, s3):
        _pool_body(x_hbm, emb_hbm, out_hbm, idx_v,
                   (r0, r1, r2, r3), out_v, (s0, s1, s2, s3))

    return pl.kernel(
        body2,
        out_type=jax.ShapeDtypeStruct((B * DIM,), jnp.float32),
        mesh=mesh,
        scratch_types=[
            pltpu.VMEM((SPB * L,), jnp.int32),
            pltpu.VMEM((L, DIM), jnp.float32),
            pltpu.VMEM((L, DIM), jnp.float32),
            pltpu.VMEM((L, DIM), jnp.float32),
            pltpu.VMEM((L, DIM), jnp.float32),
            pltpu.VMEM((SPB * DIM,), jnp.float32),
            pltpu.SemaphoreType.DMA,
            pltpu.SemaphoreType.DMA,
            pltpu.SemaphoreType.DMA,
            pltpu.SemaphoreType.DMA,
        ],
        compiler_params=pltpu.CompilerParams(use_tc_tiling_on_sc=False),
    )(x1, emb)


def _mlp_body(rep_ref, len_ref, W1_ref, b1_ref, W2_ref, b2_ref, out_ref):
    inv = 1.0 / len_ref[...].astype(jnp.float32)
    r = rep_ref[...] * inv
    h = jnp.dot(r, W1_ref[...], preferred_element_type=jnp.float32)
    h = jnp.maximum(h + b1_ref[...], 0.0)
    out_ref[...] = (jnp.dot(h, W2_ref[...], preferred_element_type=jnp.float32)
                    + b2_ref[...])


MLP_BLK = 512


def _mlp(rep, lengths2, W1, b1r, W2, b2r):
    grid = (B // MLP_BLK,)
    return pl.pallas_call(
        _mlp_body,
        grid=grid,
        in_specs=[
            pl.BlockSpec((MLP_BLK, DIM), lambda i: (i, 0)),
            pl.BlockSpec((MLP_BLK, 1), lambda i: (i, 0)),
            pl.BlockSpec((DIM, HIDDEN), lambda i: (0, 0)),
            pl.BlockSpec((1, HIDDEN), lambda i: (0, 0)),
            pl.BlockSpec((HIDDEN, OUT), lambda i: (0, 0)),
            pl.BlockSpec((1, OUT), lambda i: (0, 0)),
        ],
        out_specs=pl.BlockSpec((MLP_BLK, OUT), lambda i: (i, 0)),
        out_shape=jax.ShapeDtypeStruct((B, OUT), jnp.float32),
    )(rep, lengths2, W1, b1r, W2, b2r)


def kernel(x, lengths, emb, W1, b1, W2, b2):
    x1 = x.astype(jnp.int32).reshape(B * L)
    sums = _pool(x1, emb).reshape(B, DIM)
    return _mlp(sums, lengths.reshape(B, 1), W1, b1.reshape(1, HIDDEN),
                W2, b2.reshape(1, OUT))
